# Initial kernel scaffold; baseline (speedup 1.0000x reference)
import jax
import jax.numpy as jnp
from jax.experimental import pallas as pl


def kernel(flat, cu_seqlens, W_e, W_c):
    raise NotImplementedError("write your pallas kernel here")


# speedup vs baseline: 43.9382x; 43.9382x over previous
import jax
import jax.numpy as jnp
from jax.experimental import pallas as pl
from jax.experimental.pallas import tpu as pltpu


def _body(flat_ref, lo_ref, hi_ref, inv_ref, we_ref, wc_ref, out_ref, acc_ref,
          *, block_tok, num_blocks, num_sents):
    b = pl.program_id(0)
    t = jax.lax.broadcasted_iota(jnp.int32, (num_sents, block_tok), 1) + b * block_tok
    onehot = jnp.logical_and(t >= lo_ref[...], t < hi_ref[...]).astype(jnp.bfloat16)
    part = jax.lax.dot_general(
        onehot, flat_ref[...].astype(jnp.bfloat16),
        (((1,), (0,)), ((), ())), preferred_element_type=jnp.float32)

    @pl.when(b == 0)
    def _():
        acc_ref[...] = part

    @pl.when(b > 0)
    def _():
        acc_ref[...] += part

    @pl.when(b == num_blocks - 1)
    def _():
        mean = acc_ref[...] * inv_ref[...]
        combined = jax.lax.dot_general(
            wc_ref[...], we_ref[...], (((1,), (0,)), ((), ())),
            precision=jax.lax.Precision.HIGHEST,
            preferred_element_type=jnp.float32)
        logits = jax.lax.dot_general(
            mean, combined, (((1,), (1,)), ((), ())),
            precision=jax.lax.Precision.HIGHEST,
            preferred_element_type=jnp.float32)
        m = jnp.max(logits, axis=-1, keepdims=True)
        sh = logits - m
        lse = jnp.log(jnp.sum(jnp.exp(sh), axis=-1, keepdims=True))
        out_ref[...] = sh - lse


def kernel(flat, cu_seqlens, W_e, W_c):
    total_tok, emb = flat.shape
    num_sents = cu_seqlens.shape[0] - 1
    ntags = W_c.shape[0]
    cu = cu_seqlens.astype(jnp.int32)
    cu_lo = cu[:-1].reshape(num_sents, 1)
    cu_hi = cu[1:].reshape(num_sents, 1)
    inv = 1.0 / jnp.maximum(cu_hi - cu_lo, 1).astype(jnp.float32)

    block_tok = 2048
    num_blocks = total_tok // block_tok

    import functools
    body = functools.partial(_body, block_tok=block_tok,
                             num_blocks=num_blocks, num_sents=num_sents)

    out = pl.pallas_call(
        body,
        grid=(num_blocks,),
        in_specs=[
            pl.BlockSpec((block_tok, emb), lambda b: (b, 0)),
            pl.BlockSpec((num_sents, 1), lambda b: (0, 0)),
            pl.BlockSpec((num_sents, 1), lambda b: (0, 0)),
            pl.BlockSpec((num_sents, 1), lambda b: (0, 0)),
            pl.BlockSpec((emb, emb), lambda b: (0, 0)),
            pl.BlockSpec((ntags, emb), lambda b: (0, 0)),
        ],
        out_specs=pl.BlockSpec((num_sents, ntags), lambda b: (0, 0)),
        out_shape=jax.ShapeDtypeStruct((num_sents, ntags), jnp.float32),
        scratch_shapes=[pltpu.VMEM((num_sents, emb), jnp.float32)],
    )(flat, cu_lo, cu_hi, inv, W_e, W_c)
    return out
